# baseline (speedup 1.0000x reference)
import functools

import jax
import jax.numpy as jnp
from jax import lax
from jax.experimental import pallas as pl
from jax.experimental.pallas import tpu as pltpu
from jax.experimental.pallas import tpu_sc as plsc

N_NODES = 10000
N_EDGES = 320000
D_FEAT = 128

NC = 2
NS = 16
NW = NC * NS
E_PER_W = N_EDGES // NW
EB = 48
NB = 210
E_PAD_W = NB * EB
NSLOT = 3
N_PAD = 10240
ROWS_PER_TILE = N_PAD // NS
Z_ROWS = 128


def _sc_body(x_hbm, pk_hbm, zrows_hbm, zdeg_hbm, ones_hbm,
             acc_out, deg_out,
             pk_v, sb0, db0, sb1, db1, sb2, db2,
             buf0, buf1, buf2, ones_v, acc_sh, deg_sh,
             g0, g1, g2, s0, s1, s2, semi):
    cid = lax.axis_index("c")
    sid = lax.axis_index("s")
    wid = cid * NS + sid

    sb = (sb0, sb1, sb2)
    db = (db0, db1, db2)
    buf = (buf0, buf1, buf2)
    gsem = (g0, g1, g2)
    ssem = (s0, s1, s2)

    stripe = pl.ds(sid * ROWS_PER_TILE, ROWS_PER_TILE)
    pltpu.async_copy(pk_hbm.at[wid], pk_v, semi)
    pltpu.async_copy(ones_hbm, ones_v, semi)
    for z in range(ROWS_PER_TILE // Z_ROWS):
        pltpu.async_copy(
            zrows_hbm, acc_sh.at[pl.ds(sid * ROWS_PER_TILE + z * Z_ROWS,
                                       Z_ROWS)], semi)
    pltpu.async_copy(zdeg_hbm, deg_sh.at[stripe], semi)
    pltpu.make_async_copy(pk_hbm.at[wid], pk_v, semi).wait()
    pltpu.make_async_copy(ones_hbm, ones_v, semi).wait()
    for z in range(ROWS_PER_TILE // Z_ROWS):
        pltpu.make_async_copy(
            zrows_hbm, acc_sh.at[pl.ds(sid * ROWS_PER_TILE + z * Z_ROWS,
                                       Z_ROWS)], semi).wait()
    pltpu.make_async_copy(zdeg_hbm, deg_sh.at[stripe], semi).wait()
    plsc.subcore_barrier()

    def unpack(i, k):
        for c in range(EB // 16):
            w = pk_v[i, pl.ds(c * 16, 16)]
            sb[k][pl.ds(c * 16, 16)] = w & 0xFFFF
            db[k][pl.ds(c * 16, 16)] = lax.shift_right_logical(w, 16)

    def start_gather(k):
        pltpu.async_copy(x_hbm.at[sb[k]], buf[k], gsem[k])

    def wait_gather(k):
        pltpu.make_async_copy(x_hbm.at[sb[k]], buf[k], gsem[k]).wait()

    def start_scatters(k):
        pltpu.async_copy(buf[k], acc_sh.at[db[k]], ssem[k], add=True)
        pltpu.async_copy(ones_v, deg_sh.at[db[k]], ssem[k], add=True)

    def wait_scatters(k):
        pltpu.make_async_copy(buf[k], acc_sh.at[db[k]], ssem[k]).wait()
        pltpu.make_async_copy(ones_v, deg_sh.at[db[k]], ssem[k]).wait()

    for k in range(NSLOT):
        unpack(k, k)
        start_gather(k)

    def body(g, carry):
        i = NSLOT * g
        for k in range(NSLOT):
            wait_gather(k)
            start_scatters(k)
        for k in range(NSLOT):
            wait_scatters(k)
            unpack(i + k + NSLOT, k)
            start_gather(k)
        return carry

    lax.fori_loop(0, NB // NSLOT - 1, body, 0)

    for k in range(NSLOT):
        wait_gather(k)
        start_scatters(k)
    for k in range(NSLOT):
        wait_scatters(k)
    plsc.subcore_barrier()

    pltpu.async_copy(acc_sh.at[stripe], acc_out.at[cid, stripe], semi)
    pltpu.async_copy(deg_sh.at[stripe], deg_out.at[cid, stripe], semi)
    pltpu.make_async_copy(acc_sh.at[stripe], acc_out.at[cid, stripe],
                          semi).wait()
    pltpu.make_async_copy(deg_sh.at[stripe], deg_out.at[cid, stripe],
                          semi).wait()


@functools.partial(
    pl.kernel,
    mesh=plsc.VectorSubcoreMesh(core_axis_name="c", subcore_axis_name="s"),
    out_type=[
        jax.ShapeDtypeStruct((NC, N_PAD, D_FEAT), jnp.float32),
        jax.ShapeDtypeStruct((NC, N_PAD), jnp.float32),
    ],
    scratch_types=(
        [pltpu.VMEM((NB, EB), jnp.int32)]
        + [pltpu.VMEM((EB,), jnp.int32)] * 6
        + [pltpu.VMEM((EB, D_FEAT), jnp.float32)] * 3
        + [pltpu.VMEM((EB,), jnp.float32)]
        + [pltpu.VMEM_SHARED((N_PAD, D_FEAT), jnp.float32),
           pltpu.VMEM_SHARED((N_PAD,), jnp.float32)]
        + [pltpu.SemaphoreType.DMA] * 7
    ),
)
def _sc_aggregate(*refs):
    _sc_body(*refs)


_TC_R = 1000


def _tc_body(x_ref, p0_ref, p1_ref, d0_ref, d1_ref, ws_ref, wn_ref, b_ref,
             o_ref):
    deg = jnp.maximum(d0_ref[...] + d1_ref[...], 1.0)
    agg = (p0_ref[0] + p1_ref[0]) / deg
    h = jnp.dot(x_ref[...], ws_ref[...], preferred_element_type=jnp.float32)
    h = h + jnp.dot(agg, wn_ref[...], preferred_element_type=jnp.float32)
    o_ref[...] = jnp.maximum(h + b_ref[...], 0.0)


def _tc_finish(x, acc, d0, d1, w_self, w_neigh, b2):
    grid = (N_NODES // _TC_R,)
    row_blk = pl.BlockSpec((_TC_R, D_FEAT), lambda i: (i, 0))
    p0_blk = pl.BlockSpec((1, _TC_R, D_FEAT), lambda i: (0, i, 0))
    p1_blk = pl.BlockSpec((1, _TC_R, D_FEAT), lambda i: (1, i, 0))
    col_blk = pl.BlockSpec((_TC_R, 1), lambda i: (i, 0))
    full_w = pl.BlockSpec((D_FEAT, D_FEAT), lambda i: (0, 0))
    return pl.pallas_call(
        _tc_body,
        grid=grid,
        in_specs=[row_blk, p0_blk, p1_blk, col_blk, col_blk, full_w,
                  full_w, pl.BlockSpec((1, D_FEAT), lambda i: (0, 0))],
        out_specs=row_blk,
        out_shape=jax.ShapeDtypeStruct((N_NODES, D_FEAT), jnp.float32),
    )(x, acc, acc, d0, d1, w_self, w_neigh, b2)


def kernel(x, edge_index, batch, W_self, W_neigh, b):
    src = edge_index[0].astype(jnp.int32)
    dst = edge_index[1].astype(jnp.int32)
    packed = (src | (dst << 16)).reshape(NW, E_PER_W)
    pad = jnp.full((NW, E_PAD_W - E_PER_W), N_NODES << 16, jnp.int32)
    packed = jnp.concatenate([packed, pad], axis=1).reshape(NW, NB, EB)
    zrows = jnp.zeros((Z_ROWS, D_FEAT), jnp.float32)
    zdeg = jnp.zeros((ROWS_PER_TILE,), jnp.float32)
    ones = jnp.ones((EB,), jnp.float32)

    acc, deg = _sc_aggregate(x, packed, zrows, zdeg, ones)

    d0 = deg[0, :N_NODES].reshape(N_NODES, 1)
    d1 = deg[1, :N_NODES].reshape(N_NODES, 1)
    out = _tc_finish(x, acc, d0, d1, W_self, W_neigh, b.reshape(1, D_FEAT))
    return out, batch


# speedup vs baseline: 1.2491x; 1.2491x over previous
import functools

import jax
import jax.numpy as jnp
from jax import lax
from jax.experimental import pallas as pl
from jax.experimental.pallas import tpu as pltpu
from jax.experimental.pallas import tpu_sc as plsc

N_NODES = 10000
N_EDGES = 320000
D_FEAT = 128

NC = 2
NS = 16
NW = NC * NS
E_PER_W = N_EDGES // NW
EB = 80
NB = E_PER_W // EB
N_PAD = 10240
ROWS_PER_TILE = N_PAD // NS
Z_ROWS = 128


def _sc_body(x_hbm, pk_hbm, zrows_hbm, zdeg_hbm, ones_hbm,
             acc_out, deg_out,
             pk_v, sb0, db0, sb1, db1, buf0, buf1, ones_v, acc_sh, deg_sh,
             sem0, sem1, semd, semi):
    cid = lax.axis_index("c")
    sid = lax.axis_index("s")
    wid = cid * NS + sid

    stripe = pl.ds(sid * ROWS_PER_TILE, ROWS_PER_TILE)
    pltpu.async_copy(pk_hbm.at[wid], pk_v, semi)
    pltpu.async_copy(ones_hbm, ones_v, semi)
    for z in range(ROWS_PER_TILE // Z_ROWS):
        pltpu.async_copy(
            zrows_hbm, acc_sh.at[pl.ds(sid * ROWS_PER_TILE + z * Z_ROWS,
                                       Z_ROWS)], semi)
    pltpu.async_copy(zdeg_hbm, deg_sh.at[stripe], semi)
    pltpu.make_async_copy(pk_hbm.at[wid], pk_v, semi).wait()
    pltpu.make_async_copy(ones_hbm, ones_v, semi).wait()
    for z in range(ROWS_PER_TILE // Z_ROWS):
        pltpu.make_async_copy(
            zrows_hbm, acc_sh.at[pl.ds(sid * ROWS_PER_TILE + z * Z_ROWS,
                                       Z_ROWS)], semi).wait()
    pltpu.make_async_copy(zdeg_hbm, deg_sh.at[stripe], semi).wait()
    plsc.subcore_barrier()

    def unpack(i, src_b, dst_b):
        for k in range(EB // 16):
            w = pk_v[i, pl.ds(k * 16, 16)]
            src_b[pl.ds(k * 16, 16)] = w & 0xFFFF
            dst_b[pl.ds(k * 16, 16)] = lax.shift_right_logical(w, 16)

    def start_gather(src_b, buf, sem):
        pltpu.async_copy(x_hbm.at[src_b], buf, sem)

    def wait_gather(src_b, buf, sem):
        pltpu.make_async_copy(x_hbm.at[src_b], buf, sem).wait()

    unpack(0, sb0, db0)
    start_gather(sb0, buf0, sem0)
    unpack(1, sb1, db1)
    start_gather(sb1, buf1, sem1)

    def body(g, carry):
        i0 = 2 * g
        wait_gather(sb0, buf0, sem0)
        pltpu.sync_copy(buf0, acc_sh.at[db0], add=True)
        pltpu.async_copy(ones_v, deg_sh.at[db0], semd, add=True)
        wait_gather(sb1, buf1, sem1)
        pltpu.sync_copy(buf1, acc_sh.at[db1], add=True)
        pltpu.async_copy(ones_v, deg_sh.at[db1], semd, add=True)
        pltpu.make_async_copy(ones_v, deg_sh.at[db0], semd).wait()
        pltpu.make_async_copy(ones_v, deg_sh.at[db1], semd).wait()
        unpack(i0 + 2, sb0, db0)
        start_gather(sb0, buf0, sem0)
        unpack(i0 + 3, sb1, db1)
        start_gather(sb1, buf1, sem1)
        return carry

    lax.fori_loop(0, (NB - 3) // 2, body, 0)

    wait_gather(sb0, buf0, sem0)
    pltpu.sync_copy(buf0, acc_sh.at[db0], add=True)
    pltpu.async_copy(ones_v, deg_sh.at[db0], semd, add=True)
    wait_gather(sb1, buf1, sem1)
    pltpu.sync_copy(buf1, acc_sh.at[db1], add=True)
    pltpu.async_copy(ones_v, deg_sh.at[db1], semd, add=True)
    pltpu.make_async_copy(ones_v, deg_sh.at[db0], semd).wait()
    pltpu.make_async_copy(ones_v, deg_sh.at[db1], semd).wait()
    unpack(NB - 1, sb0, db0)
    start_gather(sb0, buf0, sem0)
    wait_gather(sb0, buf0, sem0)
    pltpu.sync_copy(buf0, acc_sh.at[db0], add=True)
    pltpu.sync_copy(ones_v, deg_sh.at[db0], add=True)
    plsc.subcore_barrier()

    pltpu.async_copy(acc_sh.at[stripe], acc_out.at[cid, stripe], semi)
    pltpu.async_copy(deg_sh.at[stripe], deg_out.at[cid, stripe], semi)
    pltpu.make_async_copy(acc_sh.at[stripe], acc_out.at[cid, stripe],
                          semi).wait()
    pltpu.make_async_copy(deg_sh.at[stripe], deg_out.at[cid, stripe],
                          semi).wait()


@functools.partial(
    pl.kernel,
    mesh=plsc.VectorSubcoreMesh(core_axis_name="c", subcore_axis_name="s"),
    out_type=[
        jax.ShapeDtypeStruct((NC, N_PAD, D_FEAT), jnp.float32),
        jax.ShapeDtypeStruct((NC, N_PAD), jnp.float32),
    ],
    scratch_types=[
        pltpu.VMEM((NB, EB), jnp.int32),
        pltpu.VMEM((EB,), jnp.int32),
        pltpu.VMEM((EB,), jnp.int32),
        pltpu.VMEM((EB,), jnp.int32),
        pltpu.VMEM((EB,), jnp.int32),
        pltpu.VMEM((EB, D_FEAT), jnp.float32),
        pltpu.VMEM((EB, D_FEAT), jnp.float32),
        pltpu.VMEM((EB,), jnp.float32),
        pltpu.VMEM_SHARED((N_PAD, D_FEAT), jnp.float32),
        pltpu.VMEM_SHARED((N_PAD,), jnp.float32),
        pltpu.SemaphoreType.DMA,
        pltpu.SemaphoreType.DMA,
        pltpu.SemaphoreType.DMA,
        pltpu.SemaphoreType.DMA,
    ],
)
def _sc_aggregate(*refs):
    _sc_body(*refs)


_TC_R = 1000


def _tc_body(x_ref, p0_ref, p1_ref, d0_ref, d1_ref, ws_ref, wn_ref, b_ref,
             o_ref):
    deg = jnp.maximum(d0_ref[...] + d1_ref[...], 1.0)
    agg = (p0_ref[0] + p1_ref[0]) / deg
    h = jnp.dot(x_ref[...], ws_ref[...], preferred_element_type=jnp.float32)
    h = h + jnp.dot(agg, wn_ref[...], preferred_element_type=jnp.float32)
    o_ref[...] = jnp.maximum(h + b_ref[...], 0.0)


def _tc_finish(x, acc, d0, d1, w_self, w_neigh, b2):
    grid = (N_NODES // _TC_R,)
    row_blk = pl.BlockSpec((_TC_R, D_FEAT), lambda i: (i, 0))
    p0_blk = pl.BlockSpec((1, _TC_R, D_FEAT), lambda i: (0, i, 0))
    p1_blk = pl.BlockSpec((1, _TC_R, D_FEAT), lambda i: (1, i, 0))
    col_blk = pl.BlockSpec((_TC_R, 1), lambda i: (i, 0))
    full_w = pl.BlockSpec((D_FEAT, D_FEAT), lambda i: (0, 0))
    return pl.pallas_call(
        _tc_body,
        grid=grid,
        in_specs=[row_blk, p0_blk, p1_blk, col_blk, col_blk, full_w,
                  full_w, pl.BlockSpec((1, D_FEAT), lambda i: (0, 0))],
        out_specs=row_blk,
        out_shape=jax.ShapeDtypeStruct((N_NODES, D_FEAT), jnp.float32),
    )(x, acc, acc, d0, d1, w_self, w_neigh, b2)


def kernel(x, edge_index, batch, W_self, W_neigh, b):
    src = edge_index[0].astype(jnp.int32)
    dst = edge_index[1].astype(jnp.int32)
    packed = (src | (dst << 16)).reshape(NW, NB, EB)
    zrows = jnp.zeros((Z_ROWS, D_FEAT), jnp.float32)
    zdeg = jnp.zeros((ROWS_PER_TILE,), jnp.float32)
    ones = jnp.ones((EB,), jnp.float32)

    acc, deg = _sc_aggregate(x, packed, zrows, zdeg, ones)

    d0 = deg[0, :N_NODES].reshape(N_NODES, 1)
    d1 = deg[1, :N_NODES].reshape(N_NODES, 1)
    out = _tc_finish(x, acc, d0, d1, W_self, W_neigh, b.reshape(1, D_FEAT))
    return out, batch


# speedup vs baseline: 1.5695x; 1.2565x over previous
import functools

import jax
import jax.numpy as jnp
from jax import lax
from jax.experimental import pallas as pl
from jax.experimental.pallas import tpu as pltpu
from jax.experimental.pallas import tpu_sc as plsc

N_NODES = 10000
N_EDGES = 320000
D_FEAT = 128

NC = 2
NS = 16
NW = NC * NS
E_PER_W = N_EDGES // NW
EB = 80
NB = E_PER_W // EB
N_PAD = 10240
ROWS_PER_TILE = N_PAD // NS
Z_ROWS = 128


def _sc_body(x_hbm, pk_hbm, zrows_hbm, zdeg_hbm, ones_hbm,
             acc_out, deg_out,
             pk_v, sb0, db0, sb1, db1, buf0, buf1, ones_v, acc_sh, deg_sh,
             sem0, sem1, s0, s1, semi):
    cid = lax.axis_index("c")
    sid = lax.axis_index("s")
    wid = cid * NS + sid

    stripe = pl.ds(sid * ROWS_PER_TILE, ROWS_PER_TILE)
    pltpu.async_copy(pk_hbm.at[wid], pk_v, semi)
    pltpu.async_copy(ones_hbm, ones_v, semi)
    for z in range(ROWS_PER_TILE // Z_ROWS):
        pltpu.async_copy(
            zrows_hbm, acc_sh.at[pl.ds(sid * ROWS_PER_TILE + z * Z_ROWS,
                                       Z_ROWS)], semi)
    pltpu.async_copy(zdeg_hbm, deg_sh.at[stripe], semi)
    pltpu.make_async_copy(pk_hbm.at[wid], pk_v, semi).wait()
    pltpu.make_async_copy(ones_hbm, ones_v, semi).wait()
    for z in range(ROWS_PER_TILE // Z_ROWS):
        pltpu.make_async_copy(
            zrows_hbm, acc_sh.at[pl.ds(sid * ROWS_PER_TILE + z * Z_ROWS,
                                       Z_ROWS)], semi).wait()
    pltpu.make_async_copy(zdeg_hbm, deg_sh.at[stripe], semi).wait()
    plsc.subcore_barrier()

    def unpack(i, src_b, dst_b):
        for k in range(EB // 16):
            w = pk_v[i, pl.ds(k * 16, 16)]
            src_b[pl.ds(k * 16, 16)] = w & 0xFFFF
            dst_b[pl.ds(k * 16, 16)] = lax.shift_right_logical(w, 16)

    def start_gather(src_b, buf, sem):
        pltpu.async_copy(x_hbm.at[src_b], buf, sem)

    def wait_gather(src_b, buf, sem):
        pltpu.make_async_copy(x_hbm.at[src_b], buf, sem).wait()

    unpack(0, sb0, db0)
    start_gather(sb0, buf0, sem0)
    unpack(1, sb1, db1)
    start_gather(sb1, buf1, sem1)

    def scat(buf, dbx, sx):
        pltpu.async_copy(buf, acc_sh.at[dbx], sx, add=True)
        pltpu.async_copy(ones_v, deg_sh.at[dbx], sx, add=True)

    def scat_wait(buf, dbx, sx):
        pltpu.make_async_copy(buf, acc_sh.at[dbx], sx).wait()
        pltpu.make_async_copy(ones_v, deg_sh.at[dbx], sx).wait()

    def body(g, carry):
        i0 = 2 * g
        wait_gather(sb0, buf0, sem0)
        scat(buf0, db0, s0)
        wait_gather(sb1, buf1, sem1)
        scat(buf1, db1, s1)
        scat_wait(buf0, db0, s0)
        unpack(i0 + 2, sb0, db0)
        start_gather(sb0, buf0, sem0)
        scat_wait(buf1, db1, s1)
        unpack(i0 + 3, sb1, db1)
        start_gather(sb1, buf1, sem1)
        return carry

    lax.fori_loop(0, (NB - 3) // 2, body, 0)

    wait_gather(sb0, buf0, sem0)
    scat(buf0, db0, s0)
    wait_gather(sb1, buf1, sem1)
    scat(buf1, db1, s1)
    scat_wait(buf0, db0, s0)
    unpack(NB - 1, sb0, db0)
    start_gather(sb0, buf0, sem0)
    scat_wait(buf1, db1, s1)
    wait_gather(sb0, buf0, sem0)
    scat(buf0, db0, s0)
    scat_wait(buf0, db0, s0)
    plsc.subcore_barrier()

    pltpu.async_copy(acc_sh.at[stripe], acc_out.at[cid, stripe], semi)
    pltpu.async_copy(deg_sh.at[stripe], deg_out.at[cid, stripe], semi)
    pltpu.make_async_copy(acc_sh.at[stripe], acc_out.at[cid, stripe],
                          semi).wait()
    pltpu.make_async_copy(deg_sh.at[stripe], deg_out.at[cid, stripe],
                          semi).wait()


@functools.partial(
    pl.kernel,
    mesh=plsc.VectorSubcoreMesh(core_axis_name="c", subcore_axis_name="s"),
    out_type=[
        jax.ShapeDtypeStruct((NC, N_PAD, D_FEAT), jnp.float32),
        jax.ShapeDtypeStruct((NC, N_PAD), jnp.float32),
    ],
    scratch_types=[
        pltpu.VMEM((NB, EB), jnp.int32),
        pltpu.VMEM((EB,), jnp.int32),
        pltpu.VMEM((EB,), jnp.int32),
        pltpu.VMEM((EB,), jnp.int32),
        pltpu.VMEM((EB,), jnp.int32),
        pltpu.VMEM((EB, D_FEAT), jnp.float32),
        pltpu.VMEM((EB, D_FEAT), jnp.float32),
        pltpu.VMEM((EB,), jnp.float32),
        pltpu.VMEM_SHARED((N_PAD, D_FEAT), jnp.float32),
        pltpu.VMEM_SHARED((N_PAD,), jnp.float32),
        pltpu.SemaphoreType.DMA,
        pltpu.SemaphoreType.DMA,
        pltpu.SemaphoreType.DMA,
        pltpu.SemaphoreType.DMA,
        pltpu.SemaphoreType.DMA,
    ],
)
def _sc_aggregate(*refs):
    _sc_body(*refs)


_TC_R = 1000


def _tc_body(x_ref, p0_ref, p1_ref, d0_ref, d1_ref, ws_ref, wn_ref, b_ref,
             o_ref):
    deg = jnp.maximum(d0_ref[...] + d1_ref[...], 1.0)
    agg = (p0_ref[0] + p1_ref[0]) / deg
    h = jnp.dot(x_ref[...], ws_ref[...], preferred_element_type=jnp.float32)
    h = h + jnp.dot(agg, wn_ref[...], preferred_element_type=jnp.float32)
    o_ref[...] = jnp.maximum(h + b_ref[...], 0.0)


def _tc_finish(x, acc, d0, d1, w_self, w_neigh, b2):
    grid = (N_NODES // _TC_R,)
    row_blk = pl.BlockSpec((_TC_R, D_FEAT), lambda i: (i, 0))
    p0_blk = pl.BlockSpec((1, _TC_R, D_FEAT), lambda i: (0, i, 0))
    p1_blk = pl.BlockSpec((1, _TC_R, D_FEAT), lambda i: (1, i, 0))
    col_blk = pl.BlockSpec((_TC_R, 1), lambda i: (i, 0))
    full_w = pl.BlockSpec((D_FEAT, D_FEAT), lambda i: (0, 0))
    return pl.pallas_call(
        _tc_body,
        grid=grid,
        in_specs=[row_blk, p0_blk, p1_blk, col_blk, col_blk, full_w,
                  full_w, pl.BlockSpec((1, D_FEAT), lambda i: (0, 0))],
        out_specs=row_blk,
        out_shape=jax.ShapeDtypeStruct((N_NODES, D_FEAT), jnp.float32),
    )(x, acc, acc, d0, d1, w_self, w_neigh, b2)


def kernel(x, edge_index, batch, W_self, W_neigh, b):
    src = edge_index[0].astype(jnp.int32)
    dst = edge_index[1].astype(jnp.int32)
    packed = (src | (dst << 16)).reshape(NW, NB, EB)
    zrows = jnp.zeros((Z_ROWS, D_FEAT), jnp.float32)
    zdeg = jnp.zeros((ROWS_PER_TILE,), jnp.float32)
    ones = jnp.ones((EB,), jnp.float32)

    acc, deg = _sc_aggregate(x, packed, zrows, zdeg, ones)

    d0 = deg[0, :N_NODES].reshape(N_NODES, 1)
    d1 = deg[1, :N_NODES].reshape(N_NODES, 1)
    out = _tc_finish(x, acc, d0, d1, W_self, W_neigh, b.reshape(1, D_FEAT))
    return out, batch
